# Initial kernel scaffold; baseline (speedup 1.0000x reference)
import jax
import jax.numpy as jnp
from jax.experimental import pallas as pl


def kernel(all_coeffs, voxels_elements, all_voxels_centroids):
    raise NotImplementedError("write your pallas kernel here")


# speedup vs baseline: 3.2955x; 3.2955x over previous
import functools

import jax
import jax.numpy as jnp
from jax import lax
from jax.experimental import pallas as pl
from jax.experimental.pallas import tpu as pltpu
from jax.experimental.pallas import tpu_sc as plsc

N_VOX = 1_000_000
N_ELEM = 100_000
N_ELEM_PAD = 100_096
D = 16
SLAB = 80
SPC = 10
CHUNK = SLAB * SPC
N_CHUNKS = N_VOX // CHUNK
GROUPS = CHUNK // 16
NW = 32


def _body(table, idx3, cent, out, sp_table, idx_v, rows_v, cent_v, out_v, sem):
    sid = lax.axis_index("s")
    wid = sid * 2 + lax.axis_index("c")
    iota = lax.iota(jnp.int32, 16)
    iota3 = iota * 3
    cols = [jnp.full((16,), m, jnp.int32) for m in range(12)]

    rows_per_sub = N_ELEM_PAD // 16
    sbase = pl.multiple_of(sid * rows_per_sub, 8)
    pltpu.sync_copy(table.at[pl.ds(sbase, rows_per_sub)],
                    sp_table.at[pl.ds(sbase, rows_per_sub)])
    plsc.subcore_barrier()

    def chunk_body(i, carry):
        c = wid + i * NW
        pltpu.sync_copy(idx3.at[c], idx_v)
        copies = [
            pltpu.async_copy(sp_table.at[idx_v.at[s]],
                             rows_v.at[pl.ds(s * SLAB, SLAB)], sem)
            for s in range(SPC)
        ]
        pltpu.sync_copy(
            cent.at[pl.ds(pl.multiple_of(c * (CHUNK * 3), 8), CHUNK * 3)],
            cent_v)
        for cp in copies:
            cp.wait()

        def g_body(g, carry2):
            ir3 = g * 48 + iota3
            irow = g * 16 + iota
            cx = plsc.load_gather(cent_v, [ir3])
            cy = plsc.load_gather(cent_v, [ir3 + 1])
            cz = plsc.load_gather(cent_v, [ir3 + 2])
            for j in range(3):
                a0 = plsc.load_gather(rows_v, [irow, cols[j]])
                a1 = plsc.load_gather(rows_v, [irow, cols[3 + j]])
                a2 = plsc.load_gather(rows_v, [irow, cols[6 + j]])
                a3 = plsc.load_gather(rows_v, [irow, cols[9 + j]])
                o = a0 + cx * a1 + cy * a2 + cz * a3
                plsc.store_scatter(out_v, [ir3 + j], o)
            return carry2

        lax.fori_loop(0, GROUPS, g_body, 0)
        pltpu.sync_copy(
            out_v,
            out.at[pl.ds(pl.multiple_of(c * (CHUNK * 3), 8), CHUNK * 3)])
        return carry

    n_my = (N_CHUNKS - wid + NW - 1) // NW
    lax.fori_loop(0, n_my, chunk_body, 0)


@functools.partial(
    pl.kernel,
    out_type=jax.ShapeDtypeStruct((N_VOX * 3,), jnp.float32),
    mesh=plsc.VectorSubcoreMesh(core_axis_name="c", subcore_axis_name="s"),
    compiler_params=pltpu.CompilerParams(needs_layout_passes=False,
                                         use_tc_tiling_on_sc=False),
    scratch_types=[
        pltpu.VMEM_SHARED((N_ELEM_PAD, D), jnp.float32),
        pltpu.VMEM((SPC, SLAB), jnp.int32),
        pltpu.VMEM((CHUNK, D), jnp.float32),
        pltpu.VMEM((CHUNK * 3,), jnp.float32),
        pltpu.VMEM((CHUNK * 3,), jnp.float32),
        pltpu.SemaphoreType.DMA,
    ],
)
def _sc_call(table, idx3, cent, out, sp_table, idx_v, rows_v, cent_v, out_v,
             sem):
    _body(table, idx3, cent, out, sp_table, idx_v, rows_v, cent_v, out_v, sem)


def kernel(all_coeffs, voxels_elements, all_voxels_centroids):
    table = jnp.pad(all_coeffs.reshape(N_ELEM, 12),
                    ((0, N_ELEM_PAD - N_ELEM), (0, D - 12)))
    idx3 = voxels_elements.reshape(N_CHUNKS, SPC, SLAB)
    flat = _sc_call(table, idx3, all_voxels_centroids.reshape(-1))
    return flat.reshape(N_VOX, 3)
